# baseline (speedup 1.0000x reference)
import functools

import jax
import jax.numpy as jnp
from jax import lax
from jax.experimental import pallas as pl
from jax.experimental.pallas import tpu as pltpu
from jax.experimental.pallas import tpu_sc as plsc

LANES = 16
NW = 32
N = 100000
B_PER_W = 3136
NV = B_PER_W // LANES
N_SPECIES = 119

_mesh = plsc.VectorSubcoreMesh(core_axis_name="c", subcore_axis_name="s")


@functools.partial(
    pl.kernel,
    mesh=_mesh,
    out_type=jax.ShapeDtypeStruct((N,), jnp.float32),
    scratch_types=[
        pltpu.VMEM((B_PER_W,), jnp.float32),
        pltpu.VMEM((B_PER_W,), jnp.int32),
        pltpu.VMEM((B_PER_W,), jnp.float32),
        pltpu.VMEM((N_SPECIES,), jnp.float32),
        pltpu.VMEM((N_SPECIES,), jnp.float32),
        pltpu.SemaphoreType.DMA,
    ],
    compiler_params=pltpu.CompilerParams(needs_layout_passes=False),
)
def _scale_shift_sc(x_hbm, z_hbm, scale_hbm, shift_hbm, out_hbm,
                    x_v, z_v, o_v, sc_v, sh_v, sem):
    wid = lax.axis_index("s") * 2 + lax.axis_index("c")
    base = jnp.minimum(wid * B_PER_W, N - B_PER_W)

    pltpu.async_copy(scale_hbm, sc_v, sem)
    pltpu.async_copy(shift_hbm, sh_v, sem)
    pltpu.async_copy(x_hbm.at[pl.ds(base, B_PER_W)], x_v, sem)
    pltpu.async_copy(z_hbm.at[pl.ds(base, B_PER_W)], z_v, sem)
    pltpu.make_async_copy(scale_hbm, sc_v, sem).wait()
    pltpu.make_async_copy(shift_hbm, sh_v, sem).wait()
    pltpu.make_async_copy(x_hbm.at[pl.ds(base, B_PER_W)], x_v, sem).wait()
    pltpu.make_async_copy(z_hbm.at[pl.ds(base, B_PER_W)], z_v, sem).wait()

    @plsc.parallel_loop(0, NV, unroll=8)
    def _(i):
        off = i * LANES
        z = z_v[pl.ds(off, LANES)]
        xv = x_v[pl.ds(off, LANES)]
        s = plsc.load_gather(sc_v, [z])
        t = plsc.load_gather(sh_v, [z])
        o_v[pl.ds(off, LANES)] = s * xv + t

    pltpu.sync_copy(o_v, out_hbm.at[pl.ds(base, B_PER_W)])


def kernel(x, Z, scale, shift):
    out = _scale_shift_sc(x.reshape(-1), Z.astype(jnp.int32),
                          scale.reshape(-1), shift.reshape(-1))
    return out.reshape(x.shape)
